# baseline (speedup 1.0000x reference)
import functools

import jax
import jax.numpy as jnp
from jax import lax
from jax.experimental import pallas as pl
from jax.experimental.pallas import tpu as pltpu
from jax.experimental.pallas import tpu_sc as plsc

BLK = 128
K_NN = 16
K_Q = 8
TD = 64
PCH = 3 * TD
PPAD = 256


def _knn_proj_kernel(xyz_blk_ref, xyz_all_ref, xt_blk_ref, wn_ref, wc_ref,
                     bctr_ref, idx_ref, tab_ref, ctr_ref, *, n_total):
    b = pl.program_id(0)
    nb = pl.program_id(1)
    base = nb * BLK

    xq = xyz_blk_ref[0]
    xa = xyz_all_ref[0]
    d2 = jnp.zeros((BLK, n_total), dtype=jnp.float32)
    for d in range(3):
        diff = xq[d][:, None] - xa[d][None, :]
        d2 = d2 + diff * diff

    iota_n = lax.broadcasted_iota(jnp.int32, (BLK, n_total), 1)
    iota_m = lax.broadcasted_iota(jnp.int32, (BLK, n_total), 0)
    big = jnp.float32(3.4e38)
    d2 = jnp.where(iota_n == iota_m + base, big, d2)

    rows = []
    for _ in range(K_NN):
        m = jnp.min(d2, axis=1)
        cand = jnp.where(d2 <= m[:, None], iota_n, jnp.int32(n_total))
        a = jnp.min(cand, axis=1)
        rows.append(a)
        d2 = jnp.where(iota_n == a[:, None], big, d2)
    idx_ref[0] = jnp.stack(rows, axis=0) + b * n_total

    xt = xt_blk_ref[0]
    dn = (((1,), (1,)), ((), ()))
    tab = lax.dot_general(xt, wn_ref[...], dn,
                          preferred_element_type=jnp.float32)
    tab_ref[0] = jnp.concatenate(
        [tab, jnp.zeros((BLK, PPAD - PCH), jnp.float32)], axis=1)
    ctr_ref[0] = lax.dot_general(xt, wc_ref[...], dn,
                                 preferred_element_type=jnp.float32) \
        + bctr_ref[...]


def _make_sc_gather(total_rows, n_workers):
    rows_per_w = total_rows // n_workers
    chunk = 128
    n_chunks = rows_per_w // chunk
    mesh = plsc.VectorSubcoreMesh(core_axis_name="c", subcore_axis_name="s")

    @functools.partial(
        pl.kernel, mesh=mesh,
        out_type=jax.ShapeDtypeStruct((total_rows, PPAD), jnp.float32),
        scratch_types=[
            pltpu.VMEM((chunk,), jnp.int32),
            pltpu.VMEM((chunk, PPAD), jnp.float32),
            pltpu.SemaphoreType.DMA,
        ],
    )
    def sc_gather(tab_hbm, idx_hbm, out_hbm, idx_v, rows_v, sem):
        wid = lax.axis_index("s") * 2 + lax.axis_index("c")
        base = wid * rows_per_w

        def body(ci, carry):
            r0 = base + ci * chunk
            pltpu.sync_copy(idx_hbm.at[pl.ds(r0, chunk)], idx_v)
            pltpu.async_copy(tab_hbm.at[idx_v], rows_v, sem).wait()
            pltpu.sync_copy(rows_v, out_hbm.at[pl.ds(r0, chunk)])
            return carry

        lax.fori_loop(0, n_chunks, body, 0)

    return sc_gather


def _attn_kernel(g_ref, ctr_ref, feat_ref, wout_ref, bc_ref, out_ref):
    g = g_ref[0]
    ctr = ctr_ref[0]
    qct = ctr[:, 0:TD]
    kct = ctr[:, TD:2 * TD]
    vct = ctr[:, 2 * TD:3 * TD]

    gq = [g[j, :, 0:TD] + qct for j in range(K_Q)]
    gk = [g[j, :, TD:2 * TD] + kct for j in range(K_NN)]
    gv = [g[j, :, 2 * TD:3 * TD] + vct for j in range(K_NN)]

    i_r = lax.broadcasted_iota(jnp.int32, (K_NN * TD, K_NN), 0)
    i_c = lax.broadcasted_iota(jnp.int32, (K_NN * TD, K_NN), 1)
    bd = (i_r // TD == i_c).astype(jnp.float32)

    w = jnp.zeros((BLK, K_NN), dtype=jnp.float32)
    for i in range(K_Q):
        e_i = jnp.concatenate([gq[i] * gk[j] for j in range(K_NN)], axis=1)
        logits = jnp.dot(e_i, bd, preferred_element_type=jnp.float32)
        mx = jnp.max(logits, axis=1, keepdims=True)
        e = jnp.exp(logits - mx)
        w = w + e / jnp.sum(e, axis=1, keepdims=True)

    res = jnp.zeros((BLK, TD), dtype=jnp.float32)
    for j in range(K_NN):
        res = res + gv[j] * w[:, j][:, None]

    out = lax.dot_general(wout_ref[...], res, (((1,), (1,)), ((), ())),
                          preferred_element_type=jnp.float32)
    out_ref[0] = out + bc_ref[...] + feat_ref[0]


def kernel(feature, xyz, Wq, bq, Wk, bk, Wv, bv, Wc, bc):
    B, C, N = feature.shape
    nblk = N // BLK
    cin = 3 + C

    def split(W):
        return W[:, 0:3], W[:, 3:6], W[:, 6:6 + C], W[:, 6 + C:6 + 2 * C]

    qa, qb, qc_, qd = split(Wq)
    ka, kb, kc_, kd = split(Wk)
    va, vb, vc_, vd = split(Wv)
    w_nbr = jnp.concatenate([
        jnp.concatenate([qa, qc_], axis=1),
        jnp.concatenate([ka, kc_], axis=1),
        jnp.concatenate([va, vc_], axis=1),
    ], axis=0)
    w_ctr = jnp.concatenate([
        jnp.concatenate([qb - qa, qd - qc_], axis=1),
        jnp.concatenate([kb - ka, kd - kc_], axis=1),
        jnp.concatenate([vb - va, vd - vc_], axis=1),
    ], axis=0)
    b_ctr = jnp.concatenate([bq, bk, bv])[None, :]
    x_t = jnp.transpose(jnp.concatenate([xyz, feature], axis=1),
                        (0, 2, 1))

    idx, tab, ctr = pl.pallas_call(
        functools.partial(_knn_proj_kernel, n_total=N),
        grid=(B, nblk),
        in_specs=[
            pl.BlockSpec((1, 3, BLK), lambda b, n: (b, 0, n)),
            pl.BlockSpec((1, 3, N), lambda b, n: (b, 0, 0)),
            pl.BlockSpec((1, BLK, cin), lambda b, n: (b, n, 0)),
            pl.BlockSpec((PCH, cin), lambda b, n: (0, 0)),
            pl.BlockSpec((PCH, cin), lambda b, n: (0, 0)),
            pl.BlockSpec((1, PCH), lambda b, n: (0, 0)),
        ],
        out_specs=[
            pl.BlockSpec((1, K_NN, BLK), lambda b, n: (b, 0, n)),
            pl.BlockSpec((1, BLK, PPAD), lambda b, n: (b, n, 0)),
            pl.BlockSpec((1, BLK, PCH), lambda b, n: (b, n, 0)),
        ],
        out_shape=[
            jax.ShapeDtypeStruct((B, K_NN, N), jnp.int32),
            jax.ShapeDtypeStruct((B, N, PPAD), jnp.float32),
            jax.ShapeDtypeStruct((B, N, PCH), jnp.float32),
        ],
    )(xyz, xyz, x_t, w_nbr, w_ctr, b_ctr)

    total_rows = B * K_NN * N
    gathered = _make_sc_gather(total_rows, 32)(
        tab.reshape(B * N, PPAD), idx.reshape(total_rows))
    g4 = gathered.reshape(B, K_NN, N, PPAD)

    out = pl.pallas_call(
        _attn_kernel,
        grid=(B, nblk),
        in_specs=[
            pl.BlockSpec((1, K_NN, BLK, PPAD), lambda b, n: (b, 0, n, 0)),
            pl.BlockSpec((1, BLK, PCH), lambda b, n: (b, n, 0)),
            pl.BlockSpec((1, C, BLK), lambda b, n: (b, 0, n)),
            pl.BlockSpec((C, TD), lambda b, n: (0, 0)),
            pl.BlockSpec((C, 1), lambda b, n: (0, 0)),
        ],
        out_specs=pl.BlockSpec((1, C, BLK), lambda b, n: (b, 0, n)),
        out_shape=jax.ShapeDtypeStruct((B, C, N), jnp.float32),
    )(g4, ctr, feature, Wc, bc[:, None])

    return out


# speedup vs baseline: 1.1729x; 1.1729x over previous
import functools

import jax
import jax.numpy as jnp
from jax import lax
from jax.experimental import pallas as pl
from jax.experimental.pallas import tpu as pltpu
from jax.experimental.pallas import tpu_sc as plsc

BLK = 128
K_NN = 16
K_Q = 8
TD = 64
PCH = 3 * TD
PPAD = 256


def _knn_proj_kernel(xyz_all_ref, xt_blk_ref, wn_ref, wc_ref,
                     bctr_ref, idx_ref, tab_ref, ctr_ref, *, n_total):
    b = pl.program_id(0)
    nb = pl.program_id(1)
    base = nb * BLK

    xt = xt_blk_ref[0]
    xa = xyz_all_ref[0]
    d2 = jnp.zeros((BLK, n_total), dtype=jnp.float32)
    for d in range(3):
        diff = xt[:, d][:, None] - xa[d][None, :]
        d2 = d2 + diff * diff

    iota_n = lax.broadcasted_iota(jnp.int32, (BLK, n_total), 1)
    iota_m = lax.broadcasted_iota(jnp.int32, (BLK, n_total), 0)
    big = jnp.float32(3.4e38)
    d2 = jnp.where(iota_n == iota_m + base, big, d2)

    rows = []
    for _ in range(K_NN):
        m = jnp.min(d2, axis=1)
        cand = jnp.where(d2 <= m[:, None], iota_n, jnp.int32(n_total))
        a = jnp.min(cand, axis=1)
        rows.append(a)
        d2 = jnp.where(iota_n == a[:, None], big, d2)
    idx_ref[0] = jnp.stack(rows, axis=0) + b * n_total

    dn = (((1,), (1,)), ((), ()))
    tab = lax.dot_general(xt, wn_ref[...], dn,
                          preferred_element_type=jnp.float32)
    tab_ref[0] = jnp.concatenate(
        [tab, jnp.zeros((BLK, PPAD - PCH), jnp.float32)], axis=1)
    ctr_ref[0] = lax.dot_general(xt, wc_ref[...], dn,
                                 preferred_element_type=jnp.float32) \
        + bctr_ref[...]


def _make_sc_gather(total_rows, n_workers):
    rows_per_w = total_rows // n_workers
    chunk = 128
    n_chunks = rows_per_w // chunk
    mesh = plsc.VectorSubcoreMesh(core_axis_name="c", subcore_axis_name="s")

    @functools.partial(
        pl.kernel, mesh=mesh,
        out_type=jax.ShapeDtypeStruct((total_rows, PPAD), jnp.float32),
        scratch_types=[
            pltpu.VMEM((chunk,), jnp.int32),
            pltpu.VMEM((chunk, PPAD), jnp.float32),
            pltpu.SemaphoreType.DMA,
        ],
    )
    def sc_gather(tab_hbm, idx_hbm, out_hbm, idx_v, rows_v, sem):
        wid = lax.axis_index("s") * 2 + lax.axis_index("c")
        base = wid * rows_per_w

        def body(ci, carry):
            r0 = base + ci * chunk
            pltpu.sync_copy(idx_hbm.at[pl.ds(r0, chunk)], idx_v)
            pltpu.async_copy(tab_hbm.at[idx_v], rows_v, sem).wait()
            pltpu.sync_copy(rows_v, out_hbm.at[pl.ds(r0, chunk)])
            return carry

        lax.fori_loop(0, n_chunks, body, 0)

    return sc_gather


def _attn_kernel(g_ref, ctr_ref, feat_ref, wout_ref, bc_ref, out_ref):
    g = g_ref[0]
    ctr = ctr_ref[0]
    qct = ctr[:, 0:TD]
    kct = ctr[:, TD:2 * TD]
    vct = ctr[:, 2 * TD:3 * TD]

    gq = [g[j, :, 0:TD] + qct for j in range(K_Q)]
    gk = [g[j, :, TD:2 * TD] + kct for j in range(K_NN)]
    gv = [g[j, :, 2 * TD:3 * TD] + vct for j in range(K_NN)]

    i_r = lax.broadcasted_iota(jnp.int32, (K_NN * TD, K_NN), 0)
    i_c = lax.broadcasted_iota(jnp.int32, (K_NN * TD, K_NN), 1)
    bd = (i_r // TD == i_c).astype(jnp.float32)

    w = jnp.zeros((BLK, K_NN), dtype=jnp.float32)
    for i in range(K_Q):
        e_i = jnp.concatenate([gq[i] * gk[j] for j in range(K_NN)], axis=1)
        logits = jnp.dot(e_i, bd, preferred_element_type=jnp.float32)
        mx = jnp.max(logits, axis=1, keepdims=True)
        e = jnp.exp(logits - mx)
        w = w + e / jnp.sum(e, axis=1, keepdims=True)

    res = jnp.zeros((BLK, TD), dtype=jnp.float32)
    for j in range(K_NN):
        res = res + gv[j] * w[:, j][:, None]

    out = lax.dot_general(wout_ref[...], res, (((1,), (1,)), ((), ())),
                          preferred_element_type=jnp.float32)
    out_ref[0] = out + bc_ref[...] + feat_ref[0]


def kernel(feature, xyz, Wq, bq, Wk, bk, Wv, bv, Wc, bc):
    B, C, N = feature.shape
    nblk = N // BLK
    cin = 3 + C

    def split(W):
        return W[:, 0:3], W[:, 3:6], W[:, 6:6 + C], W[:, 6 + C:6 + 2 * C]

    qa, qb, qc_, qd = split(Wq)
    ka, kb, kc_, kd = split(Wk)
    va, vb, vc_, vd = split(Wv)
    w_nbr = jnp.concatenate([
        jnp.concatenate([qa, qc_], axis=1),
        jnp.concatenate([ka, kc_], axis=1),
        jnp.concatenate([va, vc_], axis=1),
    ], axis=0)
    w_ctr = jnp.concatenate([
        jnp.concatenate([qb - qa, qd - qc_], axis=1),
        jnp.concatenate([kb - ka, kd - kc_], axis=1),
        jnp.concatenate([vb - va, vd - vc_], axis=1),
    ], axis=0)
    b_ctr = jnp.concatenate([bq, bk, bv])[None, :]
    x_t = jnp.transpose(jnp.concatenate([xyz, feature], axis=1),
                        (0, 2, 1))

    bh = B // 2 if B % 2 == 0 else B
    sc_gather = _make_sc_gather(bh * K_NN * N, 32)
    outs = []
    for h in range(B // bh):
        sl = slice(h * bh, (h + 1) * bh)
        idx, tab, ctr = pl.pallas_call(
            functools.partial(_knn_proj_kernel, n_total=N),
            grid=(bh, nblk),
            in_specs=[
                pl.BlockSpec((1, 3, N), lambda b, n: (b, 0, 0)),
                pl.BlockSpec((1, BLK, cin), lambda b, n: (b, n, 0)),
                pl.BlockSpec((PCH, cin), lambda b, n: (0, 0)),
                pl.BlockSpec((PCH, cin), lambda b, n: (0, 0)),
                pl.BlockSpec((1, PCH), lambda b, n: (0, 0)),
            ],
            out_specs=[
                pl.BlockSpec((1, K_NN, BLK), lambda b, n: (b, 0, n)),
                pl.BlockSpec((1, BLK, PPAD), lambda b, n: (b, n, 0)),
                pl.BlockSpec((1, BLK, PCH), lambda b, n: (b, n, 0)),
            ],
            out_shape=[
                jax.ShapeDtypeStruct((bh, K_NN, N), jnp.int32),
                jax.ShapeDtypeStruct((bh, N, PPAD), jnp.float32),
                jax.ShapeDtypeStruct((bh, N, PCH), jnp.float32),
            ],
        )(xyz[sl], x_t[sl], w_nbr, w_ctr, b_ctr)

        gathered = sc_gather(tab.reshape(bh * N, PPAD),
                             idx.reshape(bh * K_NN * N))
        g4 = gathered.reshape(bh, K_NN, N, PPAD)

        outs.append(pl.pallas_call(
            _attn_kernel,
            grid=(bh, nblk),
            in_specs=[
                pl.BlockSpec((1, K_NN, BLK, PPAD),
                             lambda b, n: (b, 0, n, 0)),
                pl.BlockSpec((1, BLK, PCH), lambda b, n: (b, n, 0)),
                pl.BlockSpec((1, C, BLK), lambda b, n: (b, 0, n)),
                pl.BlockSpec((C, TD), lambda b, n: (0, 0)),
                pl.BlockSpec((C, 1), lambda b, n: (0, 0)),
            ],
            out_specs=pl.BlockSpec((1, C, BLK), lambda b, n: (b, 0, n)),
            out_shape=jax.ShapeDtypeStruct((bh, C, N), jnp.float32),
        )(g4, ctr, feature[sl], Wc, bc[:, None]))

    return outs[0] if len(outs) == 1 else jnp.concatenate(outs, axis=0)
